# baseline (speedup 1.0000x reference)
import functools

import jax
import jax.numpy as jnp
from jax import lax
from jax.experimental import pallas as pl
from jax.experimental.pallas import tpu as pltpu
from jax.experimental.pallas import tpu_sc as plsc

_NC = 2
_NS = 16
_NW = _NC * _NS
_LANES = 16
_CH = 128


def _make_seg_sum(N, H, B):
    segs_per_w = B // _NW
    lanes_per_row = H // _LANES
    mesh = plsc.VectorSubcoreMesh(core_axis_name="c", subcore_axis_name="s")

    @functools.partial(
        pl.kernel,
        mesh=mesh,
        out_type=jax.ShapeDtypeStruct((B * H,), jnp.float32),
        scratch_types=[
            pltpu.VMEM((B,), jnp.int32),
            pltpu.VMEM((B + 16,), jnp.int32),
            pltpu.VMEM((32,), jnp.int32),
            pltpu.VMEM((_CH, H), jnp.float32),
            pltpu.VMEM((_CH, H), jnp.float32),
            pltpu.VMEM((segs_per_w * H,), jnp.float32),
            pltpu.SemaphoreType.DMA,
            pltpu.SemaphoreType.DMA,
        ],
    )
    def seg_sum(emb_hbm, nv_hbm, out_hbm, nvv, offv, scanb, buf0, buf1, outv,
                sem0, sem1):
        wid = lax.axis_index("s") * _NC + lax.axis_index("c")

        pltpu.sync_copy(nv_hbm, nvv)
        scanb[pl.ds(0, 16)] = jnp.zeros((16,), jnp.int32)
        running = jnp.int32(0)
        for b in range(B // 16):
            v = nvv[pl.ds(b * 16, 16)]
            cs = v
            for d in (1, 2, 4, 8):
                scanb[pl.ds(16, 16)] = cs
                cs = cs + scanb[pl.ds(16 - d, 16)]
            offv[pl.ds(b * 16, 16)] = (cs - v) + running
            running = running + cs[15]
        offv[pl.ds(B, 16)] = jnp.full((16,), running, dtype=jnp.int32)
        bv = offv[pl.ds(segs_per_w * wid, 16)]
        bounds = [bv[j] for j in range(segs_per_w + 1)]

        def start(k, buf, sem, a):
            o = pl.multiple_of(jnp.minimum(a + k * _CH, N - _CH), 8)
            pltpu.async_copy(emb_hbm.at[pl.ds(o, _CH)], buf, sem)

        def wait(buf, sem):
            pltpu.make_async_copy(emb_hbm.at[pl.ds(0, _CH)], buf, sem).wait()

        for j in range(segs_per_w):
            s = bounds[j]
            e = bounds[j + 1]
            a = (s // 8) * 8
            nch = (e - a + _CH - 1) // _CH
            npairs = (nch + 1) // 2

            def accum(buf, k, accs, s=s, e=e, a=a):
                lo = a + k * _CH
                o = jnp.minimum(lo, N - _CH)

                def row_body(r, accs):
                    g = o + r
                    valid = (g >= lo) & (g >= s) & (g < e)
                    mv = jnp.full((16,), jnp.where(valid, 1.0, 0.0),
                                  dtype=jnp.float32)
                    return tuple(
                        accs[l] + buf[r, pl.ds(l * 16, 16)] * mv
                        for l in range(lanes_per_row)
                    )

                return lax.fori_loop(0, _CH, row_body, accs)

            start(0, buf0, sem0, a)
            start(1, buf1, sem1, a)

            def pair_body(p, accs, a=a):
                wait(buf0, sem0)
                accs = accum(buf0, 2 * p, accs)
                start(2 * p + 2, buf0, sem0, a)
                wait(buf1, sem1)
                accs = accum(buf1, 2 * p + 1, accs)
                start(2 * p + 3, buf1, sem1, a)
                return accs

            accs = lax.fori_loop(
                0, jnp.maximum(npairs - 1, 0), pair_body,
                tuple(jnp.zeros((16,), jnp.float32)
                      for _ in range(lanes_per_row)))
            wait(buf0, sem0)
            accs = accum(buf0, 2 * npairs - 2, accs)
            wait(buf1, sem1)
            accs = accum(buf1, 2 * npairs - 1, accs)

            for l in range(lanes_per_row):
                outv[pl.ds(j * H + l * 16, 16)] = accs[l]

        pltpu.sync_copy(
            outv, out_hbm.at[pl.ds(wid * segs_per_w * H, segs_per_w * H)])

    return seg_sum


def _mlp_body(s_ref, c_ref, w1_ref, b1_ref, w2_ref, b2_ref, w3_ref, b3_ref,
              o_ref):
    cnt = jnp.maximum(c_ref[...], 1.0)
    x = s_ref[...] / cnt
    h = jnp.dot(x, w1_ref[...], preferred_element_type=jnp.float32)
    h = jnp.maximum(h + b1_ref[...], 0.0)
    h = jnp.dot(h, w2_ref[...], preferred_element_type=jnp.float32)
    h = jnp.maximum(h + b2_ref[...], 0.0)
    o = jnp.sum(h * w3_ref[...], axis=1, keepdims=True) + b3_ref[...]
    o_ref[...] = 1.0 / (1.0 + jnp.exp(-o))


def kernel(node_embedding, W1, b1, W2, b2, W3, b3, node_type, num_variable):
    N, H = node_embedding.shape
    B = num_variable.shape[0]

    sums = _make_seg_sum(N, H, B)(node_embedding, num_variable)
    sums = sums.reshape(B, H)

    out = pl.pallas_call(
        _mlp_body,
        out_shape=jax.ShapeDtypeStruct((B, 1), jnp.float32),
    )(
        sums,
        num_variable.astype(jnp.float32).reshape(B, 1),
        W1, b1.reshape(1, H),
        W2, b2.reshape(1, H),
        W3.reshape(1, H),
        b3.reshape(1, 1),
    )
    return out.reshape(B)


# speedup vs baseline: 1.0930x; 1.0930x over previous
import functools

import jax
import jax.numpy as jnp
from jax import lax
from jax.experimental import pallas as pl
from jax.experimental.pallas import tpu as pltpu
from jax.experimental.pallas import tpu_sc as plsc

_NC = 2
_NS = 16
_NW = _NC * _NS
_LANES = 16
_CH = 112


def _make_seg_sum(N, H, B):
    segs_per_w = B // _NW
    lanes_per_row = H // _LANES
    mesh = plsc.VectorSubcoreMesh(core_axis_name="c", subcore_axis_name="s")

    @functools.partial(
        pl.kernel,
        mesh=mesh,
        out_type=jax.ShapeDtypeStruct((B * H,), jnp.float32),
        scratch_types=[
            pltpu.VMEM((B,), jnp.int32),
            pltpu.VMEM((B + 16,), jnp.int32),
            pltpu.VMEM((32,), jnp.int32),
            pltpu.VMEM((_CH, H), jnp.float32),
            pltpu.VMEM((_CH, H), jnp.float32),
            pltpu.VMEM((_CH, H), jnp.float32),
            pltpu.VMEM((_CH, H), jnp.float32),
            pltpu.VMEM((segs_per_w * H,), jnp.float32),
            pltpu.SemaphoreType.DMA,
            pltpu.SemaphoreType.DMA,
            pltpu.SemaphoreType.DMA,
            pltpu.SemaphoreType.DMA,
        ],
    )
    def seg_sum(emb_hbm, nv_hbm, out_hbm, nvv, offv, scanb,
                bufa0, bufa1, bufb0, bufb1, outv, sema0, sema1, semb0, semb1):
        wid = lax.axis_index("s") * _NC + lax.axis_index("c")

        pltpu.sync_copy(nv_hbm, nvv)
        scanb[pl.ds(0, 16)] = jnp.zeros((16,), jnp.int32)
        running = jnp.int32(0)
        for b in range(B // 16):
            v = nvv[pl.ds(b * 16, 16)]
            cs = v
            for d in (1, 2, 4, 8):
                scanb[pl.ds(16, 16)] = cs
                cs = cs + scanb[pl.ds(16 - d, 16)]
            offv[pl.ds(b * 16, 16)] = (cs - v) + running
            running = running + cs[15]
        offv[pl.ds(B, 16)] = jnp.full((16,), running, dtype=jnp.int32)
        bv = offv[pl.ds(segs_per_w * wid, 16)]
        bounds = [bv[j] for j in range(segs_per_w + 1)]

        pairs = ((bufa0, bufa1, sema0, sema1), (bufb0, bufb1, semb0, semb1))

        def start(k, buf, sem, a):
            o = pl.multiple_of(jnp.minimum(a + k * _CH, N - _CH), 8)
            pltpu.async_copy(emb_hbm.at[pl.ds(o, _CH)], buf, sem)

        def wait(buf, sem):
            pltpu.make_async_copy(emb_hbm.at[pl.ds(0, _CH)], buf, sem).wait()

        def seg_params(j):
            s = bounds[j]
            e = bounds[j + 1]
            a = (s // 8) * 8
            nch = (e - a + _CH - 1) // _CH
            return s, e, a, (nch + 1) // 2

        def prologue(j):
            _, _, a, _ = seg_params(j)
            b0, b1, s0, s1 = pairs[j % 2]
            start(0, b0, s0, a)
            start(1, b1, s1, a)

        def run_segment(j):
            s, e, a, npairs = seg_params(j)
            b0, b1, s0, s1 = pairs[j % 2]

            def accum(buf, k, accs):
                lo = a + k * _CH
                o = jnp.minimum(lo, N - _CH)

                def row_body(r, accs):
                    g = o + r
                    valid = (g >= lo) & (g >= s) & (g < e)
                    mv = jnp.full((16,), jnp.where(valid, 1.0, 0.0),
                                  dtype=jnp.float32)
                    return tuple(
                        accs[l] + buf[r, pl.ds(l * 16, 16)] * mv
                        for l in range(lanes_per_row)
                    )

                return lax.fori_loop(0, _CH, row_body, accs)

            def pair_body(p, accs):
                wait(b0, s0)
                accs = accum(b0, 2 * p, accs)
                start(2 * p + 2, b0, s0, a)
                wait(b1, s1)
                accs = accum(b1, 2 * p + 1, accs)
                start(2 * p + 3, b1, s1, a)
                return accs

            accs = lax.fori_loop(
                0, jnp.maximum(npairs - 1, 0), pair_body,
                tuple(jnp.zeros((16,), jnp.float32)
                      for _ in range(lanes_per_row)))
            wait(b0, s0)
            accs = accum(b0, 2 * npairs - 2, accs)
            wait(b1, s1)
            accs = accum(b1, 2 * npairs - 1, accs)

            cnt = jnp.full((16,), e - s, dtype=jnp.int32)
            scale = 1.0 / jnp.maximum(cnt.astype(jnp.float32), 1.0)
            for l in range(lanes_per_row):
                outv[pl.ds(j * H + l * 16, 16)] = accs[l] * scale

        prologue(0)
        prologue(1)
        for j in range(segs_per_w):
            run_segment(j)
            if j + 2 < segs_per_w:
                prologue(j + 2)

        pltpu.sync_copy(
            outv, out_hbm.at[pl.ds(wid * segs_per_w * H, segs_per_w * H)])

    return seg_sum


def _mlp_body(s_ref, w1_ref, b1_ref, w2_ref, b2_ref, w3_ref, b3_ref, o_ref):
    x = s_ref[...]
    h = jnp.dot(x, w1_ref[...], preferred_element_type=jnp.float32)
    h = jnp.maximum(h + b1_ref[...], 0.0)
    h = jnp.dot(h, w2_ref[...], preferred_element_type=jnp.float32)
    h = jnp.maximum(h + b2_ref[...], 0.0)
    o = jnp.sum(h * w3_ref[...], axis=1, keepdims=True) + b3_ref[...]
    o_ref[...] = 1.0 / (1.0 + jnp.exp(-o))


def kernel(node_embedding, W1, b1, W2, b2, W3, b3, node_type, num_variable):
    N, H = node_embedding.shape
    B = num_variable.shape[0]

    means = _make_seg_sum(N, H, B)(node_embedding, num_variable)
    means = means.reshape(B, H)

    out = pl.pallas_call(
        _mlp_body,
        out_shape=jax.ShapeDtypeStruct((B, 1), jnp.float32),
    )(
        means,
        W1, b1.reshape(1, H),
        W2, b2.reshape(1, H),
        W3.reshape(1, H),
        b3.reshape(1, 1),
    )
    return out.reshape(B)


# speedup vs baseline: 1.1286x; 1.0325x over previous
import functools

import jax
import jax.numpy as jnp
from jax import lax
from jax.experimental import pallas as pl
from jax.experimental.pallas import tpu as pltpu
from jax.experimental.pallas import tpu_sc as plsc

_NC = 2
_NS = 16
_NW = _NC * _NS
_LANES = 16
_CH = 112


def _make_seg_sum(N, H, B):
    segs_per_w = B // _NW
    lanes_per_row = H // _LANES
    mesh = plsc.VectorSubcoreMesh(core_axis_name="c", subcore_axis_name="s")

    @functools.partial(
        pl.kernel,
        mesh=mesh,
        out_type=jax.ShapeDtypeStruct((B * H,), jnp.float32),
        scratch_types=[
            pltpu.VMEM((B,), jnp.int32),
            pltpu.VMEM((B + 16,), jnp.int32),
            pltpu.VMEM((32,), jnp.int32),
            pltpu.VMEM((_CH, H), jnp.float32),
            pltpu.VMEM((_CH, H), jnp.float32),
            pltpu.VMEM((_CH, H), jnp.float32),
            pltpu.VMEM((_CH, H), jnp.float32),
            pltpu.VMEM((segs_per_w * H,), jnp.float32),
            pltpu.SemaphoreType.DMA,
            pltpu.SemaphoreType.DMA,
            pltpu.SemaphoreType.DMA,
            pltpu.SemaphoreType.DMA,
        ],
    )
    def seg_sum(emb_hbm, nv_hbm, out_hbm, nvv, offv, scanb,
                bufa0, bufa1, bufb0, bufb1, outv, sema0, sema1, semb0, semb1):
        wid = lax.axis_index("s") * _NC + lax.axis_index("c")

        pltpu.sync_copy(nv_hbm, nvv)
        scanb[pl.ds(0, 16)] = jnp.zeros((16,), jnp.int32)
        running = jnp.int32(0)
        for b in range(B // 16):
            v = nvv[pl.ds(b * 16, 16)]
            cs = v
            for d in (1, 2, 4, 8):
                scanb[pl.ds(16, 16)] = cs
                cs = cs + scanb[pl.ds(16 - d, 16)]
            offv[pl.ds(b * 16, 16)] = (cs - v) + running
            running = running + cs[15]
        offv[pl.ds(B, 16)] = jnp.full((16,), running, dtype=jnp.int32)
        bv = offv[pl.ds(segs_per_w * wid, 16)]
        bounds = [bv[j] for j in range(segs_per_w + 1)]

        pairs = ((bufa0, bufa1, sema0, sema1), (bufb0, bufb1, semb0, semb1))

        def start(k, buf, sem, a):
            o = pl.multiple_of(jnp.minimum(a + k * _CH, N - _CH), 8)
            pltpu.async_copy(emb_hbm.at[pl.ds(o, _CH)], buf, sem)

        def wait(buf, sem):
            pltpu.make_async_copy(emb_hbm.at[pl.ds(0, _CH)], buf, sem).wait()

        def seg_params(j):
            s = bounds[j]
            e = bounds[j + 1]
            a = (s // 8) * 8
            nch = (e - a + _CH - 1) // _CH
            return s, e, a, (nch + 1) // 2

        def prologue(j):
            _, _, a, _ = seg_params(j)
            b0, b1, s0, s1 = pairs[j % 2]
            start(0, b0, s0, a)
            start(1, b1, s1, a)

        def run_segment(j):
            s, e, a, npairs = seg_params(j)
            b0, b1, s0, s1 = pairs[j % 2]

            def accum(buf, k, accs):
                lo = a + k * _CH
                o = jnp.minimum(lo, N - _CH)

                def row_body(r, accs):
                    g = o + r
                    valid = (g >= lo) & (g >= s) & (g < e)
                    mv = jnp.full((16,), jnp.where(valid, 1.0, 0.0),
                                  dtype=jnp.float32)
                    return tuple(
                        accs[l] + buf[r, pl.ds(l * 16, 16)] * mv
                        for l in range(lanes_per_row)
                    )

                return lax.fori_loop(0, _CH, row_body, accs)

            def pair_body(p, accs):
                wait(b0, s0)
                accs = accum(b0, 2 * p, accs)
                start(2 * p + 2, b0, s0, a)
                wait(b1, s1)
                accs = accum(b1, 2 * p + 1, accs)
                start(2 * p + 3, b1, s1, a)
                return accs

            accs = lax.fori_loop(
                0, jnp.maximum(npairs - 1, 0), pair_body,
                tuple(jnp.zeros((16,), jnp.float32)
                      for _ in range(lanes_per_row)))
            wait(b0, s0)
            accs = accum(b0, 2 * npairs - 2, accs)
            wait(b1, s1)
            accs = accum(b1, 2 * npairs - 1, accs)

            cnt = jnp.full((16,), e - s, dtype=jnp.int32)
            scale = 1.0 / jnp.maximum(cnt.astype(jnp.float32), 1.0)
            for l in range(lanes_per_row):
                outv[pl.ds(j * H + l * 16, 16)] = accs[l] * scale

        prologue(0)
        prologue(1)
        for j in range(segs_per_w):
            run_segment(j)
            if j + 2 < segs_per_w:
                prologue(j + 2)

        pltpu.sync_copy(
            outv, out_hbm.at[pl.ds(wid * segs_per_w * H, segs_per_w * H)])

    return seg_sum


def _mlp_body(s_ref, w1_ref, b1_ref, w2_ref, b2_ref, w3_ref, b3_ref, o_ref):
    B = o_ref.shape[0]
    x = s_ref[...].reshape(B, -1)
    h = jnp.dot(x, w1_ref[...], preferred_element_type=jnp.float32)
    h = jnp.maximum(h + b1_ref[...], 0.0)
    h = jnp.dot(h, w2_ref[...], preferred_element_type=jnp.float32)
    h = jnp.maximum(h + b2_ref[...], 0.0)
    o = jnp.sum(h * w3_ref[...], axis=1, keepdims=True) + b3_ref[...]
    o_ref[...] = 1.0 / (1.0 + jnp.exp(-o))


def kernel(node_embedding, W1, b1, W2, b2, W3, b3, node_type, num_variable):
    N, H = node_embedding.shape
    B = num_variable.shape[0]

    means = _make_seg_sum(N, H, B)(node_embedding, num_variable)

    out = pl.pallas_call(
        _mlp_body,
        out_shape=jax.ShapeDtypeStruct((B, 1), jnp.float32),
    )(
        means,
        W1, b1.reshape(1, H),
        W2, b2.reshape(1, H),
        W3.reshape(1, H),
        b3.reshape(1, 1),
    )
    return out.reshape(B)
